# baseline (speedup 1.0000x reference)
import jax
import jax.numpy as jnp
from jax import lax
from jax.experimental import pallas as pl
from jax.experimental.pallas import tpu as pltpu
from jax.experimental.pallas import tpu_sc as plsc

N_NODES = 10000
N_EDGES = 320000
IN_DIM = 128
OUT_DIM = 128

NC = 2
NS = 16
NW = NC * NS
E_PER_W = N_EDGES // NW
CHUNK = 80
N_CHUNKS = E_PER_W // CHUNK
N_PAD = 10240
ROWS_PER_TILE = N_PAD // NS
DEPTH = 3
NGRP = CHUNK // 16
NVR = OUT_DIM // 16
WIN = 256
SBLK = 2000
RED_COLS = 80


def _prep_body(x_ref, wfc_ref, wa_ref, z_ref, s_ref):
    z = jnp.dot(x_ref[...], wfc_ref[...], preferred_element_type=jnp.float32)
    z_ref[...] = z
    s_ref[...] = jnp.dot(z, wa_ref[...], preferred_element_type=jnp.float32)


def _prep(x, w_fc, w_attn2):
    blk = 2000
    grid = N_NODES // blk
    return pl.pallas_call(
        _prep_body,
        grid=(grid,),
        in_specs=[
            pl.BlockSpec((blk, IN_DIM), lambda i: (i, 0)),
            pl.BlockSpec((IN_DIM, OUT_DIM), lambda i: (0, 0)),
            pl.BlockSpec((OUT_DIM, 2), lambda i: (0, 0)),
        ],
        out_specs=[
            pl.BlockSpec((blk, OUT_DIM), lambda i: (i, 0)),
            pl.BlockSpec((blk, 2), lambda i: (i, 0)),
        ],
        out_shape=[
            jax.ShapeDtypeStruct((N_NODES, OUT_DIM), jnp.float32),
            jax.ShapeDtypeStruct((N_NODES, 2), jnp.float32),
        ],
    )(x, w_fc, w_attn2)


def _scores_body(s1_hbm, s2_hbm, src_hbm, dst_hbm, w_hbm, den_hbm,
                 s1_v, s2_v, sblk, dblk, wstg, denom_v, dred,
                 acc_sem, dstage):
    c = lax.axis_index("c")
    s = lax.axis_index("s")
    wid = c * NS + s
    zeros16 = jnp.zeros((16,), jnp.float32)

    pltpu.sync_copy(s1_hbm, s1_v)
    pltpu.sync_copy(s2_hbm, s2_v)

    def _zero_d(i, carry):
        denom_v[pl.ds(i * 16, 16)] = zeros16
        return carry

    lax.fori_loop(0, N_PAD // 16, _zero_d, 0)

    for r in range(E_PER_W // SBLK):
        off = wid * E_PER_W + r * SBLK
        pltpu.sync_copy(src_hbm.at[pl.ds(off, SBLK)], sblk)
        pltpu.sync_copy(dst_hbm.at[pl.ds(off, SBLK)], dblk)

        def _grp(g, carry):
            srcg = sblk[pl.ds(g * 16, 16)]
            dstg = dblk[pl.ds(g * 16, 16)]
            e = plsc.load_gather(s1_v, [srcg]) + plsc.load_gather(s2_v, [dstg])
            e = jnp.where(e > 0, e, e * jnp.float32(0.01))
            w16 = jnp.exp(e)
            wstg[pl.ds(g * 16, 16)] = w16
            plsc.addupdate_scatter(denom_v, [dstg], w16)
            return carry

        lax.fori_loop(0, SBLK // 16, _grp, 0)
        pltpu.sync_copy(wstg, w_hbm.at[pl.ds(off, SBLK)])

    pltpu.sync_copy(denom_v, dstage.at[s])
    plsc.subcore_barrier()
    for r in range(ROWS_PER_TILE // RED_COLS):
        col0 = s * ROWS_PER_TILE + r * RED_COLS
        pltpu.sync_copy(dstage.at[:, pl.ds(col0, RED_COLS)], dred)
        for v in range(RED_COLS // 16):
            acc16 = dred[0, pl.ds(v * 16, 16)]
            for t in range(1, NS):
                acc16 = acc16 + dred[t, pl.ds(v * 16, 16)]
            denom_v[pl.ds(r * RED_COLS + v * 16, 16)] = acc16
    pltpu.sync_copy(denom_v.at[pl.ds(0, ROWS_PER_TILE)],
                    den_hbm.at[pl.ds(c * N_PAD + s * ROWS_PER_TILE,
                                     ROWS_PER_TILE)])


def _scores(s1, s2, src, dst):
    mesh = plsc.VectorSubcoreMesh(core_axis_name="c", subcore_axis_name="s",
                                  num_cores=NC, num_subcores=NS)
    return pl.kernel(
        _scores_body,
        out_type=[
            jax.ShapeDtypeStruct((N_EDGES,), jnp.float32),
            jax.ShapeDtypeStruct((NC * N_PAD,), jnp.float32),
        ],
        mesh=mesh,
        compiler_params=pltpu.CompilerParams(needs_layout_passes=False,
                                             use_tc_tiling_on_sc=False),
        scratch_types=[
            pltpu.VMEM((N_NODES,), jnp.float32),
            pltpu.VMEM((N_NODES,), jnp.float32),
            pltpu.VMEM((SBLK,), jnp.int32),
            pltpu.VMEM((SBLK,), jnp.int32),
            pltpu.VMEM((SBLK,), jnp.float32),
            pltpu.VMEM((N_PAD,), jnp.float32),
            pltpu.VMEM((NS, RED_COLS), jnp.float32),
            pltpu.SemaphoreType.DMA,
            pltpu.VMEM_SHARED((NS, N_PAD), jnp.float32),
        ],
    )(s1, s2, src, dst)


def _rows_body(z_hbm, src_hbm, dst_hbm, w_hbm, out_hbm,
               wsrc0, wsrc1, wsrc2, wdst0, wdst1, wdst2, ww0, ww1, ww2,
               isrc0, isrc1, isrc2, sdst0, sdst1, sdst2, wb0, wb1, wb2,
               zr0, zr1, zr2,
               wsem0, wsem1, wsem2, gsem0, gsem1, gsem2, csem0, csem1, csem2,
               acc_sh):
    c = lax.axis_index("c")
    s = lax.axis_index("s")
    wid = c * NS + s
    wsrcs = [wsrc0, wsrc1, wsrc2]
    wdsts = [wdst0, wdst1, wdst2]
    wws = [ww0, ww1, ww2]
    isrcs = [isrc0, isrc1, isrc2]
    sdsts = [sdst0, sdst1, sdst2]
    wbs = [wb0, wb1, wb2]
    zrs = [zr0, zr1, zr2]
    wsems = [wsem0, wsem1, wsem2]
    gsems = [gsem0, gsem1, gsem2]
    csems = [csem0, csem1, csem2]
    zeros16 = jnp.zeros((16,), jnp.float32)

    def _zero_row(i, carry):
        for g in range(NVR):
            zr0[i, pl.ds(g * 16, 16)] = zeros16
        return carry

    lax.fori_loop(0, CHUNK, _zero_row, 0)
    for q in range(ROWS_PER_TILE // CHUNK):
        pltpu.sync_copy(zr0,
                        acc_sh.at[pl.ds(s * ROWS_PER_TILE + q * CHUNK, CHUNK)])
    plsc.subcore_barrier()

    def _wbase(cj):
        off = wid * E_PER_W + cj * CHUNK
        return off, jnp.minimum((off // 128) * 128, N_EDGES - WIN)

    def fetch_win(cj, k):
        off, base = _wbase(cj)
        pltpu.async_copy(src_hbm.at[pl.ds(base, WIN)], wsrcs[k], wsems[k])
        pltpu.async_copy(dst_hbm.at[pl.ds(base, WIN)], wdsts[k], wsems[k])
        pltpu.async_copy(w_hbm.at[pl.ds(base, WIN)], wws[k], wsems[k])

    def wait_win(cj, k):
        off, base = _wbase(cj)
        pltpu.make_async_copy(src_hbm.at[pl.ds(base, WIN)], wsrcs[k],
                              wsems[k]).wait()
        pltpu.make_async_copy(dst_hbm.at[pl.ds(base, WIN)], wdsts[k],
                              wsems[k]).wait()
        pltpu.make_async_copy(w_hbm.at[pl.ds(base, WIN)], wws[k],
                              wsems[k]).wait()

    def extract(cj, k):
        off, base = _wbase(cj)
        d = off - base
        for g in range(NGRP):
            isrcs[k][pl.ds(g * 16, 16)] = wsrcs[k][pl.ds(d + g * 16, 16)]
            sdsts[k][pl.ds(g * 16, 16)] = wdsts[k][pl.ds(d + g * 16, 16)]
            wbs[k][pl.ds(g * 16, 16)] = wws[k][pl.ds(d + g * 16, 16)]

    def issue(k):
        pltpu.async_copy(z_hbm.at[isrcs[k]], zrs[k], gsems[k])

    def drain_gather(k):
        pltpu.make_async_copy(z_hbm.at[isrcs[k]], zrs[k], gsems[k]).wait()

    def drain_scatter(k):
        pltpu.make_async_copy(zrs[k], acc_sh.at[sdsts[k]], csems[k]).wait()

    def process(k):
        drain_gather(k)

        def _group(gi, inner):
            w16 = wbs[k][pl.ds(gi * 16, 16)]
            for e16 in range(16):
                e = gi * 16 + e16
                w = w16[e16]
                for g in range(NVR):
                    zrs[k][e, pl.ds(g * 16, 16)] = (
                        zrs[k][e, pl.ds(g * 16, 16)] * w)
            return inner

        lax.fori_loop(0, NGRP, _group, 0)
        pltpu.async_copy(zrs[k], acc_sh.at[sdsts[k]], csems[k], add=True)

    def phase(cj, k, k2, guard_lo):
        process(k)

        @pl.when(cj + DEPTH < N_CHUNKS)
        def _fetch():
            fetch_win(cj + DEPTH, k)

        cn = cj + 2

        @pl.when(cn < N_CHUNKS)
        def _stage():
            if guard_lo:
                @pl.when(cn >= DEPTH)
                def _d():
                    drain_scatter(k2)
            else:
                drain_scatter(k2)
            wait_win(cn, k2)
            extract(cn, k2)
            issue(k2)

    for k in range(DEPTH):
        fetch_win(k, k)
    wait_win(0, 0)
    extract(0, 0)
    issue(0)
    wait_win(1, 1)
    extract(1, 1)
    issue(1)

    def _iter(i, carry):
        for k in range(DEPTH):
            phase(i * DEPTH + k, k, (k + 2) % DEPTH, guard_lo=True)
        return carry

    lax.fori_loop(0, (N_CHUNKS - 2) // DEPTH, _iter, 0)
    phase(N_CHUNKS - 2, 0, 2, guard_lo=False)
    phase(N_CHUNKS - 1, 1, 0, guard_lo=False)
    drain_scatter(2)
    drain_scatter(0)
    drain_scatter(1)
    plsc.subcore_barrier()

    nq = ROWS_PER_TILE // CHUNK

    def _r0(q):
        return s * ROWS_PER_TILE + q * CHUNK

    for q in range(nq):
        k = q % 2
        if q >= 2:
            pltpu.make_async_copy(zrs[k],
                                  out_hbm.at[c].at[pl.ds(_r0(q - 2), CHUNK)],
                                  csems[k]).wait()
        pltpu.sync_copy(acc_sh.at[pl.ds(_r0(q), CHUNK)], zrs[k])
        pltpu.async_copy(zrs[k], out_hbm.at[c].at[pl.ds(_r0(q), CHUNK)],
                         csems[k])
    for q in range(nq - 2, nq):
        k = q % 2
        pltpu.make_async_copy(zrs[k],
                              out_hbm.at[c].at[pl.ds(_r0(q), CHUNK)],
                              csems[k]).wait()


def _rows(z, src, dst, w_flat):
    mesh = plsc.VectorSubcoreMesh(core_axis_name="c", subcore_axis_name="s",
                                  num_cores=NC, num_subcores=NS)
    win_i = pltpu.VMEM((WIN,), jnp.int32)
    win_f = pltpu.VMEM((WIN,), jnp.float32)
    idx = pltpu.VMEM((CHUNK,), jnp.int32)
    wbuf = pltpu.VMEM((CHUNK,), jnp.float32)
    zbuf = pltpu.VMEM((CHUNK, OUT_DIM), jnp.float32)
    sem = pltpu.SemaphoreType.DMA
    return pl.kernel(
        _rows_body,
        out_type=jax.ShapeDtypeStruct((NC, N_PAD, OUT_DIM), jnp.float32),
        mesh=mesh,
        compiler_params=pltpu.CompilerParams(needs_layout_passes=False),
        scratch_types=(
            [win_i] * 3 + [win_i] * 3 + [win_f] * 3 +
            [idx] * 3 + [idx] * 3 + [wbuf] * 3 + [zbuf] * 3 +
            [sem] * 9 +
            [pltpu.VMEM_SHARED((N_PAD, OUT_DIM), jnp.float32)]
        ),
    )(z, src, dst, w_flat)


def _final_body(acc_ref, den_ref, out_ref):
    p = acc_ref[0] + acc_ref[1]
    den = den_ref[0] + den_ref[1]
    h = p / jnp.maximum(den, 1e-20)
    out_ref[...] = jnp.where(h > 0, h, jnp.exp(h) - 1.0)


def _final(acc, den):
    blk = 2000
    grid = N_NODES // blk
    return pl.pallas_call(
        _final_body,
        grid=(grid,),
        in_specs=[
            pl.BlockSpec((NC, blk, OUT_DIM), lambda i: (0, i, 0)),
            pl.BlockSpec((NC, blk, 1), lambda i: (0, i, 0)),
        ],
        out_specs=pl.BlockSpec((blk, OUT_DIM), lambda i: (i, 0)),
        out_shape=jax.ShapeDtypeStruct((N_NODES, OUT_DIM), jnp.float32),
    )(acc, den)


def kernel(x, edge_index, W_fc, W_attn):
    edge_index = edge_index.astype(jnp.int32)
    src = edge_index[0]
    dst = edge_index[1]
    w_attn2 = jnp.concatenate(
        [W_attn[:OUT_DIM], W_attn[OUT_DIM:]], axis=1)
    z, s12 = _prep(x, W_fc, w_attn2)
    w_flat, den = _scores(s12[:, 0], s12[:, 1], src, dst)
    acc = _rows(z, src, dst, w_flat)
    return _final(acc, den.reshape(NC, N_PAD, 1))


# speedup vs baseline: 1.0352x; 1.0352x over previous
import jax
import jax.numpy as jnp
from jax import lax
from jax.experimental import pallas as pl
from jax.experimental.pallas import tpu as pltpu
from jax.experimental.pallas import tpu_sc as plsc

N_NODES = 10000
N_EDGES = 320000
IN_DIM = 128
OUT_DIM = 128

NC = 2
NS = 16
NW = NC * NS
E_PER_W = N_EDGES // NW
CHUNK = 80
N_CHUNKS = E_PER_W // CHUNK
N_PAD = 10240
ROWS_PER_TILE = N_PAD // NS
DEPTH = 3
NGRP = CHUNK // 16
NVR = OUT_DIM // 16
WIN = 256
SBLK = 2000
RED_COLS = 80


def _prep_body(x_ref, wfc_ref, wa_ref, z_ref, s_ref):
    z = jnp.dot(x_ref[...], wfc_ref[...], preferred_element_type=jnp.float32)
    z_ref[...] = z
    s_ref[...] = jnp.dot(z, wa_ref[...], preferred_element_type=jnp.float32)


def _prep(x, w_fc, w_attn2):
    blk = 2000
    grid = N_NODES // blk
    return pl.pallas_call(
        _prep_body,
        grid=(grid,),
        in_specs=[
            pl.BlockSpec((blk, IN_DIM), lambda i: (i, 0)),
            pl.BlockSpec((IN_DIM, OUT_DIM), lambda i: (0, 0)),
            pl.BlockSpec((OUT_DIM, 2), lambda i: (0, 0)),
        ],
        out_specs=[
            pl.BlockSpec((blk, OUT_DIM), lambda i: (i, 0)),
            pl.BlockSpec((blk, 2), lambda i: (i, 0)),
        ],
        out_shape=[
            jax.ShapeDtypeStruct((N_NODES, OUT_DIM), jnp.float32),
            jax.ShapeDtypeStruct((N_NODES, 2), jnp.float32),
        ],
    )(x, w_fc, w_attn2)


def _scores_body(s1_hbm, s2_hbm, ei_hbm, w_hbm, src_out, dst_out, den_hbm,
                 s1_v, s2_v, sblk, dblk, wstg, denom_v, dred,
                 acc_sem, dstage):
    c = lax.axis_index("c")
    s = lax.axis_index("s")
    wid = c * NS + s
    zeros16 = jnp.zeros((16,), jnp.float32)

    pltpu.sync_copy(s1_hbm, s1_v)
    pltpu.sync_copy(s2_hbm, s2_v)

    def _zero_d(i, carry):
        denom_v[pl.ds(i * 16, 16)] = zeros16
        return carry

    lax.fori_loop(0, N_PAD // 16, _zero_d, 0)

    for r in range(E_PER_W // SBLK):
        off = wid * E_PER_W + r * SBLK
        pltpu.sync_copy(ei_hbm.at[0].at[pl.ds(off, SBLK)], sblk)
        pltpu.sync_copy(ei_hbm.at[1].at[pl.ds(off, SBLK)], dblk)

        def _grp(g, carry):
            srcg = sblk[pl.ds(g * 16, 16)]
            dstg = dblk[pl.ds(g * 16, 16)]
            e = plsc.load_gather(s1_v, [srcg]) + plsc.load_gather(s2_v, [dstg])
            e = jnp.where(e > 0, e, e * jnp.float32(0.01))
            w16 = jnp.exp(e)
            wstg[pl.ds(g * 16, 16)] = w16
            plsc.addupdate_scatter(denom_v, [dstg], w16)
            return carry

        lax.fori_loop(0, SBLK // 16, _grp, 0)
        pltpu.sync_copy(wstg, w_hbm.at[pl.ds(off, SBLK)])
        pltpu.sync_copy(sblk, src_out.at[pl.ds(off, SBLK)])
        pltpu.sync_copy(dblk, dst_out.at[pl.ds(off, SBLK)])

    pltpu.sync_copy(denom_v, dstage.at[s])
    plsc.subcore_barrier()
    for r in range(ROWS_PER_TILE // RED_COLS):
        col0 = s * ROWS_PER_TILE + r * RED_COLS
        pltpu.sync_copy(dstage.at[:, pl.ds(col0, RED_COLS)], dred)
        for v in range(RED_COLS // 16):
            acc16 = dred[0, pl.ds(v * 16, 16)]
            for t in range(1, NS):
                acc16 = acc16 + dred[t, pl.ds(v * 16, 16)]
            denom_v[pl.ds(r * RED_COLS + v * 16, 16)] = acc16
    pltpu.sync_copy(denom_v.at[pl.ds(0, ROWS_PER_TILE)],
                    den_hbm.at[pl.ds(c * N_PAD + s * ROWS_PER_TILE,
                                     ROWS_PER_TILE)])


def _scores(s1, s2, ei):
    mesh = plsc.VectorSubcoreMesh(core_axis_name="c", subcore_axis_name="s",
                                  num_cores=NC, num_subcores=NS)
    return pl.kernel(
        _scores_body,
        out_type=[
            jax.ShapeDtypeStruct((N_EDGES,), jnp.float32),
            jax.ShapeDtypeStruct((N_EDGES,), jnp.int32),
            jax.ShapeDtypeStruct((N_EDGES,), jnp.int32),
            jax.ShapeDtypeStruct((NC * N_PAD,), jnp.float32),
        ],
        mesh=mesh,
        compiler_params=pltpu.CompilerParams(needs_layout_passes=False,
                                             use_tc_tiling_on_sc=False),
        scratch_types=[
            pltpu.VMEM((N_NODES,), jnp.float32),
            pltpu.VMEM((N_NODES,), jnp.float32),
            pltpu.VMEM((SBLK,), jnp.int32),
            pltpu.VMEM((SBLK,), jnp.int32),
            pltpu.VMEM((SBLK,), jnp.float32),
            pltpu.VMEM((N_PAD,), jnp.float32),
            pltpu.VMEM((NS, RED_COLS), jnp.float32),
            pltpu.SemaphoreType.DMA,
            pltpu.VMEM_SHARED((NS, N_PAD), jnp.float32),
        ],
    )(s1, s2, ei)


def _rows_body(z_hbm, src_hbm, dst_hbm, w_hbm, out_hbm,
               wsrc0, wsrc1, wsrc2, wdst0, wdst1, wdst2, ww0, ww1, ww2,
               isrc0, isrc1, isrc2, sdst0, sdst1, sdst2, wb0, wb1, wb2,
               zr0, zr1, zr2,
               wsem0, wsem1, wsem2, gsem0, gsem1, gsem2, csem0, csem1, csem2,
               acc_sh):
    c = lax.axis_index("c")
    s = lax.axis_index("s")
    wid = c * NS + s
    wsrcs = [wsrc0, wsrc1, wsrc2]
    wdsts = [wdst0, wdst1, wdst2]
    wws = [ww0, ww1, ww2]
    isrcs = [isrc0, isrc1, isrc2]
    sdsts = [sdst0, sdst1, sdst2]
    wbs = [wb0, wb1, wb2]
    zrs = [zr0, zr1, zr2]
    wsems = [wsem0, wsem1, wsem2]
    gsems = [gsem0, gsem1, gsem2]
    csems = [csem0, csem1, csem2]
    zeros16 = jnp.zeros((16,), jnp.float32)

    def _zero_row(i, carry):
        for g in range(NVR):
            zr0[i, pl.ds(g * 16, 16)] = zeros16
        return carry

    lax.fori_loop(0, CHUNK, _zero_row, 0)
    for q in range(ROWS_PER_TILE // CHUNK):
        pltpu.sync_copy(zr0,
                        acc_sh.at[pl.ds(s * ROWS_PER_TILE + q * CHUNK, CHUNK)])
    plsc.subcore_barrier()

    def _wbase(cj):
        off = wid * E_PER_W + cj * CHUNK
        return off, jnp.minimum((off // 128) * 128, N_EDGES - WIN)

    def fetch_win(cj, k):
        off, base = _wbase(cj)
        pltpu.async_copy(src_hbm.at[pl.ds(base, WIN)], wsrcs[k], wsems[k])
        pltpu.async_copy(dst_hbm.at[pl.ds(base, WIN)], wdsts[k], wsems[k])
        pltpu.async_copy(w_hbm.at[pl.ds(base, WIN)], wws[k], wsems[k])

    def wait_win(cj, k):
        off, base = _wbase(cj)
        pltpu.make_async_copy(src_hbm.at[pl.ds(base, WIN)], wsrcs[k],
                              wsems[k]).wait()
        pltpu.make_async_copy(dst_hbm.at[pl.ds(base, WIN)], wdsts[k],
                              wsems[k]).wait()
        pltpu.make_async_copy(w_hbm.at[pl.ds(base, WIN)], wws[k],
                              wsems[k]).wait()

    def extract(cj, k):
        off, base = _wbase(cj)
        d = off - base
        for g in range(NGRP):
            isrcs[k][pl.ds(g * 16, 16)] = wsrcs[k][pl.ds(d + g * 16, 16)]
            sdsts[k][pl.ds(g * 16, 16)] = wdsts[k][pl.ds(d + g * 16, 16)]
            wbs[k][pl.ds(g * 16, 16)] = wws[k][pl.ds(d + g * 16, 16)]

    def issue(k):
        pltpu.async_copy(z_hbm.at[isrcs[k]], zrs[k], gsems[k])

    def drain_gather(k):
        pltpu.make_async_copy(z_hbm.at[isrcs[k]], zrs[k], gsems[k]).wait()

    def drain_scatter(k):
        pltpu.make_async_copy(zrs[k], acc_sh.at[sdsts[k]], csems[k]).wait()

    def process(k):
        drain_gather(k)

        def _group(gi, inner):
            w16 = wbs[k][pl.ds(gi * 16, 16)]
            for e16 in range(16):
                e = gi * 16 + e16
                w = w16[e16]
                for g in range(NVR):
                    zrs[k][e, pl.ds(g * 16, 16)] = (
                        zrs[k][e, pl.ds(g * 16, 16)] * w)
            return inner

        lax.fori_loop(0, NGRP, _group, 0)
        pltpu.async_copy(zrs[k], acc_sh.at[sdsts[k]], csems[k], add=True)

    def phase(cj, k, k2, guard_lo):
        process(k)

        @pl.when(cj + DEPTH < N_CHUNKS)
        def _fetch():
            fetch_win(cj + DEPTH, k)

        cn = cj + 2

        @pl.when(cn < N_CHUNKS)
        def _stage():
            if guard_lo:
                @pl.when(cn >= DEPTH)
                def _d():
                    drain_scatter(k2)
            else:
                drain_scatter(k2)
            wait_win(cn, k2)
            extract(cn, k2)
            issue(k2)

    for k in range(DEPTH):
        fetch_win(k, k)
    wait_win(0, 0)
    extract(0, 0)
    issue(0)
    wait_win(1, 1)
    extract(1, 1)
    issue(1)

    def _iter(i, carry):
        for k in range(DEPTH):
            phase(i * DEPTH + k, k, (k + 2) % DEPTH, guard_lo=True)
        return carry

    lax.fori_loop(0, (N_CHUNKS - 2) // DEPTH, _iter, 0)
    phase(N_CHUNKS - 2, 0, 2, guard_lo=False)
    phase(N_CHUNKS - 1, 1, 0, guard_lo=False)
    drain_scatter(2)
    drain_scatter(0)
    drain_scatter(1)
    plsc.subcore_barrier()

    nq = ROWS_PER_TILE // CHUNK

    def _r0(q):
        return s * ROWS_PER_TILE + q * CHUNK

    for q in range(nq):
        k = q % 2
        if q >= 2:
            pltpu.make_async_copy(zrs[k],
                                  out_hbm.at[c].at[pl.ds(_r0(q - 2), CHUNK)],
                                  csems[k]).wait()
        pltpu.sync_copy(acc_sh.at[pl.ds(_r0(q), CHUNK)], zrs[k])
        pltpu.async_copy(zrs[k], out_hbm.at[c].at[pl.ds(_r0(q), CHUNK)],
                         csems[k])
    for q in range(nq - 2, nq):
        k = q % 2
        pltpu.make_async_copy(zrs[k],
                              out_hbm.at[c].at[pl.ds(_r0(q), CHUNK)],
                              csems[k]).wait()


def _rows(z, src, dst, w_flat):
    mesh = plsc.VectorSubcoreMesh(core_axis_name="c", subcore_axis_name="s",
                                  num_cores=NC, num_subcores=NS)
    win_i = pltpu.VMEM((WIN,), jnp.int32)
    win_f = pltpu.VMEM((WIN,), jnp.float32)
    idx = pltpu.VMEM((CHUNK,), jnp.int32)
    wbuf = pltpu.VMEM((CHUNK,), jnp.float32)
    zbuf = pltpu.VMEM((CHUNK, OUT_DIM), jnp.float32)
    sem = pltpu.SemaphoreType.DMA
    return pl.kernel(
        _rows_body,
        out_type=jax.ShapeDtypeStruct((NC, N_PAD, OUT_DIM), jnp.float32),
        mesh=mesh,
        compiler_params=pltpu.CompilerParams(needs_layout_passes=False),
        scratch_types=(
            [win_i] * 3 + [win_i] * 3 + [win_f] * 3 +
            [idx] * 3 + [idx] * 3 + [wbuf] * 3 + [zbuf] * 3 +
            [sem] * 9 +
            [pltpu.VMEM_SHARED((N_PAD, OUT_DIM), jnp.float32)]
        ),
    )(z, src, dst, w_flat)


def _final_body(acc_ref, den_ref, out_ref):
    p = acc_ref[0] + acc_ref[1]
    den = den_ref[0] + den_ref[1]
    h = p / jnp.maximum(den, 1e-20)
    out_ref[...] = jnp.where(h > 0, h, jnp.exp(h) - 1.0)


def _final(acc, den):
    blk = 2000
    grid = N_NODES // blk
    return pl.pallas_call(
        _final_body,
        grid=(grid,),
        in_specs=[
            pl.BlockSpec((NC, blk, OUT_DIM), lambda i: (0, i, 0)),
            pl.BlockSpec((NC, blk, 1), lambda i: (0, i, 0)),
        ],
        out_specs=pl.BlockSpec((blk, OUT_DIM), lambda i: (i, 0)),
        out_shape=jax.ShapeDtypeStruct((N_NODES, OUT_DIM), jnp.float32),
    )(acc, den)


def kernel(x, edge_index, W_fc, W_attn):
    edge_index = edge_index.astype(jnp.int32)
    w_attn2 = jnp.concatenate(
        [W_attn[:OUT_DIM], W_attn[OUT_DIM:]], axis=1)
    z, s12 = _prep(x, W_fc, w_attn2)
    w_flat, src, dst, den = _scores(s12[:, 0], s12[:, 1], edge_index)
    acc = _rows(z, src, dst, w_flat)
    return _final(acc, den.reshape(NC, N_PAD, 1))


# speedup vs baseline: 1.0618x; 1.0256x over previous
import jax
import jax.numpy as jnp
from jax import lax
from jax.experimental import pallas as pl
from jax.experimental.pallas import tpu as pltpu
from jax.experimental.pallas import tpu_sc as plsc

N_NODES = 10000
N_EDGES = 320000
IN_DIM = 128
OUT_DIM = 128

NC = 2
NS = 16
NW = NC * NS
E_PER_W = N_EDGES // NW
CHUNK = 80
N_CHUNKS = E_PER_W // CHUNK
N_PAD = 10240
ROWS_PER_TILE = N_PAD // NS
DEPTH = 3
NGRP = CHUNK // 16
NVR = OUT_DIM // 16
WIN = 256
SBLK = 400
RED_COLS = 80


def _prep_body(x_ref, wfc_ref, wa_ref, z_ref, s_ref):
    z = jnp.dot(x_ref[...], wfc_ref[...], preferred_element_type=jnp.float32)
    z_ref[...] = z
    s_ref[...] = jnp.dot(z, wa_ref[...], preferred_element_type=jnp.float32)


def _prep(x, w_fc, w_attn2):
    blk = 2000
    grid = N_NODES // blk
    return pl.pallas_call(
        _prep_body,
        grid=(grid,),
        in_specs=[
            pl.BlockSpec((blk, IN_DIM), lambda i: (i, 0)),
            pl.BlockSpec((IN_DIM, OUT_DIM), lambda i: (0, 0)),
            pl.BlockSpec((OUT_DIM, 2), lambda i: (0, 0)),
        ],
        out_specs=[
            pl.BlockSpec((blk, OUT_DIM), lambda i: (i, 0)),
            pl.BlockSpec((blk, 2), lambda i: (i, 0)),
        ],
        out_shape=[
            jax.ShapeDtypeStruct((N_NODES, OUT_DIM), jnp.float32),
            jax.ShapeDtypeStruct((N_NODES, 2), jnp.float32),
        ],
    )(x, w_fc, w_attn2)


NBLK = E_PER_W // SBLK


def _scores_body(s1_hbm, s2_hbm, ei_hbm, w_hbm, src_out, dst_out, den_hbm,
                 s1_v, s2_v, sblk0, sblk1, sblk2, dblk0, dblk1, dblk2,
                 wstg0, wstg1, wstg2, denom_v, dred,
                 isem0, isem1, isem2, osem0, osem1, osem2, dstage):
    c = lax.axis_index("c")
    s = lax.axis_index("s")
    wid = c * NS + s
    sblks = [sblk0, sblk1, sblk2]
    dblks = [dblk0, dblk1, dblk2]
    wstgs = [wstg0, wstg1, wstg2]
    isems = [isem0, isem1, isem2]
    osems = [osem0, osem1, osem2]
    zeros16 = jnp.zeros((16,), jnp.float32)

    def _off(r):
        return wid * E_PER_W + r * SBLK

    def fetch(r, k):
        pltpu.async_copy(ei_hbm.at[0].at[pl.ds(_off(r), SBLK)], sblks[k],
                         isems[k])
        pltpu.async_copy(ei_hbm.at[1].at[pl.ds(_off(r), SBLK)], dblks[k],
                         isems[k])

    def wait_idx(r, k):
        pltpu.make_async_copy(ei_hbm.at[0].at[pl.ds(_off(r), SBLK)], sblks[k],
                              isems[k]).wait()
        pltpu.make_async_copy(ei_hbm.at[1].at[pl.ds(_off(r), SBLK)], dblks[k],
                              isems[k]).wait()

    def issue_writes(r, k):
        pltpu.async_copy(wstgs[k], w_hbm.at[pl.ds(_off(r), SBLK)], osems[k])
        pltpu.async_copy(sblks[k], src_out.at[pl.ds(_off(r), SBLK)], osems[k])
        pltpu.async_copy(dblks[k], dst_out.at[pl.ds(_off(r), SBLK)], osems[k])

    def drain_writes(r):
        k = r % 3
        pltpu.make_async_copy(wstgs[k], w_hbm.at[pl.ds(_off(r), SBLK)],
                              osems[k]).wait()
        pltpu.make_async_copy(sblks[k], src_out.at[pl.ds(_off(r), SBLK)],
                              osems[k]).wait()
        pltpu.make_async_copy(dblks[k], dst_out.at[pl.ds(_off(r), SBLK)],
                              osems[k]).wait()

    fetch(0, 0)
    pltpu.sync_copy(s1_hbm, s1_v)
    pltpu.sync_copy(s2_hbm, s2_v)

    def _zero_d(i, carry):
        denom_v[pl.ds(i * 16, 16)] = zeros16
        return carry

    lax.fori_loop(0, N_PAD // 16, _zero_d, 0)

    for r in range(NBLK):
        k = r % 3
        kn = (r + 1) % 3
        if r + 1 < NBLK:
            if r >= 2:
                drain_writes(r - 2)
            fetch(r + 1, kn)
        wait_idx(r, k)

        def _grp(g, carry, k=k):
            srcg = sblks[k][pl.ds(g * 16, 16)]
            dstg = dblks[k][pl.ds(g * 16, 16)]
            e = plsc.load_gather(s1_v, [srcg]) + plsc.load_gather(s2_v, [dstg])
            e = jnp.where(e > 0, e, e * jnp.float32(0.01))
            w16 = jnp.exp(e)
            wstgs[k][pl.ds(g * 16, 16)] = w16
            plsc.addupdate_scatter(denom_v, [dstg], w16)
            return carry

        lax.fori_loop(0, SBLK // 16, _grp, 0)
        issue_writes(r, k)
    for r in range(NBLK - 3, NBLK):
        drain_writes(r)

    pltpu.sync_copy(denom_v, dstage.at[s])
    plsc.subcore_barrier()
    for r in range(ROWS_PER_TILE // RED_COLS):
        col0 = s * ROWS_PER_TILE + r * RED_COLS
        pltpu.sync_copy(dstage.at[:, pl.ds(col0, RED_COLS)], dred)
        for v in range(RED_COLS // 16):
            acc16 = dred[0, pl.ds(v * 16, 16)]
            for t in range(1, NS):
                acc16 = acc16 + dred[t, pl.ds(v * 16, 16)]
            denom_v[pl.ds(r * RED_COLS + v * 16, 16)] = acc16
    pltpu.sync_copy(denom_v.at[pl.ds(0, ROWS_PER_TILE)],
                    den_hbm.at[pl.ds(c * N_PAD + s * ROWS_PER_TILE,
                                     ROWS_PER_TILE)])


def _scores(s1, s2, ei):
    mesh = plsc.VectorSubcoreMesh(core_axis_name="c", subcore_axis_name="s",
                                  num_cores=NC, num_subcores=NS)
    iblk = pltpu.VMEM((SBLK,), jnp.int32)
    fblk = pltpu.VMEM((SBLK,), jnp.float32)
    sem = pltpu.SemaphoreType.DMA
    return pl.kernel(
        _scores_body,
        out_type=[
            jax.ShapeDtypeStruct((N_EDGES,), jnp.float32),
            jax.ShapeDtypeStruct((N_EDGES,), jnp.int32),
            jax.ShapeDtypeStruct((N_EDGES,), jnp.int32),
            jax.ShapeDtypeStruct((NC * N_PAD,), jnp.float32),
        ],
        mesh=mesh,
        compiler_params=pltpu.CompilerParams(needs_layout_passes=False,
                                             use_tc_tiling_on_sc=False),
        scratch_types=(
            [pltpu.VMEM((N_NODES,), jnp.float32)] * 2 +
            [iblk] * 6 + [fblk] * 3 +
            [pltpu.VMEM((N_PAD,), jnp.float32)] +
            [pltpu.VMEM((NS, RED_COLS), jnp.float32)] +
            [sem] * 6 +
            [pltpu.VMEM_SHARED((NS, N_PAD), jnp.float32)]
        ),
    )(s1, s2, ei)


def _rows_body(z_hbm, src_hbm, dst_hbm, w_hbm, out_hbm,
               wsrc0, wsrc1, wsrc2, wdst0, wdst1, wdst2, ww0, ww1, ww2,
               isrc0, isrc1, isrc2, sdst0, sdst1, sdst2, wb0, wb1, wb2,
               zr0, zr1, zr2,
               wsem0, wsem1, wsem2, gsem0, gsem1, gsem2, csem0, csem1, csem2,
               acc_sh):
    c = lax.axis_index("c")
    s = lax.axis_index("s")
    wid = c * NS + s
    wsrcs = [wsrc0, wsrc1, wsrc2]
    wdsts = [wdst0, wdst1, wdst2]
    wws = [ww0, ww1, ww2]
    isrcs = [isrc0, isrc1, isrc2]
    sdsts = [sdst0, sdst1, sdst2]
    wbs = [wb0, wb1, wb2]
    zrs = [zr0, zr1, zr2]
    wsems = [wsem0, wsem1, wsem2]
    gsems = [gsem0, gsem1, gsem2]
    csems = [csem0, csem1, csem2]
    zeros16 = jnp.zeros((16,), jnp.float32)

    def _zero_row(i, carry):
        for g in range(NVR):
            zr0[i, pl.ds(g * 16, 16)] = zeros16
        return carry

    lax.fori_loop(0, CHUNK, _zero_row, 0)
    for q in range(ROWS_PER_TILE // CHUNK):
        pltpu.sync_copy(zr0,
                        acc_sh.at[pl.ds(s * ROWS_PER_TILE + q * CHUNK, CHUNK)])
    plsc.subcore_barrier()

    def _wbase(cj):
        off = wid * E_PER_W + cj * CHUNK
        return off, jnp.minimum((off // 128) * 128, N_EDGES - WIN)

    def fetch_win(cj, k):
        off, base = _wbase(cj)
        pltpu.async_copy(src_hbm.at[pl.ds(base, WIN)], wsrcs[k], wsems[k])
        pltpu.async_copy(dst_hbm.at[pl.ds(base, WIN)], wdsts[k], wsems[k])
        pltpu.async_copy(w_hbm.at[pl.ds(base, WIN)], wws[k], wsems[k])

    def wait_win(cj, k):
        off, base = _wbase(cj)
        pltpu.make_async_copy(src_hbm.at[pl.ds(base, WIN)], wsrcs[k],
                              wsems[k]).wait()
        pltpu.make_async_copy(dst_hbm.at[pl.ds(base, WIN)], wdsts[k],
                              wsems[k]).wait()
        pltpu.make_async_copy(w_hbm.at[pl.ds(base, WIN)], wws[k],
                              wsems[k]).wait()

    def extract(cj, k):
        off, base = _wbase(cj)
        d = off - base
        for g in range(NGRP):
            isrcs[k][pl.ds(g * 16, 16)] = wsrcs[k][pl.ds(d + g * 16, 16)]
            sdsts[k][pl.ds(g * 16, 16)] = wdsts[k][pl.ds(d + g * 16, 16)]
            wbs[k][pl.ds(g * 16, 16)] = wws[k][pl.ds(d + g * 16, 16)]

    def issue(k):
        pltpu.async_copy(z_hbm.at[isrcs[k]], zrs[k], gsems[k])

    def drain_gather(k):
        pltpu.make_async_copy(z_hbm.at[isrcs[k]], zrs[k], gsems[k]).wait()

    def drain_scatter(k):
        pltpu.make_async_copy(zrs[k], acc_sh.at[sdsts[k]], csems[k]).wait()

    def process(k):
        drain_gather(k)

        def _group(gi, inner):
            w16 = wbs[k][pl.ds(gi * 16, 16)]
            for e16 in range(16):
                e = gi * 16 + e16
                w = w16[e16]
                for g in range(NVR):
                    zrs[k][e, pl.ds(g * 16, 16)] = (
                        zrs[k][e, pl.ds(g * 16, 16)] * w)
            return inner

        lax.fori_loop(0, NGRP, _group, 0)
        pltpu.async_copy(zrs[k], acc_sh.at[sdsts[k]], csems[k], add=True)

    def phase(cj, k, k2, guard_lo):
        process(k)

        @pl.when(cj + DEPTH < N_CHUNKS)
        def _fetch():
            fetch_win(cj + DEPTH, k)

        cn = cj + 2

        @pl.when(cn < N_CHUNKS)
        def _stage():
            if guard_lo:
                @pl.when(cn >= DEPTH)
                def _d():
                    drain_scatter(k2)
            else:
                drain_scatter(k2)
            wait_win(cn, k2)
            extract(cn, k2)
            issue(k2)

    for k in range(DEPTH):
        fetch_win(k, k)
    wait_win(0, 0)
    extract(0, 0)
    issue(0)
    wait_win(1, 1)
    extract(1, 1)
    issue(1)

    def _iter(i, carry):
        for k in range(DEPTH):
            phase(i * DEPTH + k, k, (k + 2) % DEPTH, guard_lo=True)
        return carry

    lax.fori_loop(0, (N_CHUNKS - 2) // DEPTH, _iter, 0)
    phase(N_CHUNKS - 2, 0, 2, guard_lo=False)
    phase(N_CHUNKS - 1, 1, 0, guard_lo=False)
    drain_scatter(2)
    drain_scatter(0)
    drain_scatter(1)
    plsc.subcore_barrier()

    nq = ROWS_PER_TILE // CHUNK

    def _r0(q):
        return s * ROWS_PER_TILE + q * CHUNK

    for q in range(nq):
        k = q % 2
        if q >= 2:
            pltpu.make_async_copy(zrs[k],
                                  out_hbm.at[c].at[pl.ds(_r0(q - 2), CHUNK)],
                                  csems[k]).wait()
        pltpu.sync_copy(acc_sh.at[pl.ds(_r0(q), CHUNK)], zrs[k])
        pltpu.async_copy(zrs[k], out_hbm.at[c].at[pl.ds(_r0(q), CHUNK)],
                         csems[k])
    for q in range(nq - 2, nq):
        k = q % 2
        pltpu.make_async_copy(zrs[k],
                              out_hbm.at[c].at[pl.ds(_r0(q), CHUNK)],
                              csems[k]).wait()


def _rows(z, src, dst, w_flat):
    mesh = plsc.VectorSubcoreMesh(core_axis_name="c", subcore_axis_name="s",
                                  num_cores=NC, num_subcores=NS)
    win_i = pltpu.VMEM((WIN,), jnp.int32)
    win_f = pltpu.VMEM((WIN,), jnp.float32)
    idx = pltpu.VMEM((CHUNK,), jnp.int32)
    wbuf = pltpu.VMEM((CHUNK,), jnp.float32)
    zbuf = pltpu.VMEM((CHUNK, OUT_DIM), jnp.float32)
    sem = pltpu.SemaphoreType.DMA
    return pl.kernel(
        _rows_body,
        out_type=jax.ShapeDtypeStruct((NC, N_PAD, OUT_DIM), jnp.float32),
        mesh=mesh,
        compiler_params=pltpu.CompilerParams(needs_layout_passes=False),
        scratch_types=(
            [win_i] * 3 + [win_i] * 3 + [win_f] * 3 +
            [idx] * 3 + [idx] * 3 + [wbuf] * 3 + [zbuf] * 3 +
            [sem] * 9 +
            [pltpu.VMEM_SHARED((N_PAD, OUT_DIM), jnp.float32)]
        ),
    )(z, src, dst, w_flat)


def _final_body(acc_ref, den_ref, out_ref):
    p = acc_ref[0] + acc_ref[1]
    den = den_ref[0] + den_ref[1]
    h = p / jnp.maximum(den, 1e-20)
    out_ref[...] = jnp.where(h > 0, h, jnp.exp(h) - 1.0)


def _final(acc, den):
    blk = 2000
    grid = N_NODES // blk
    return pl.pallas_call(
        _final_body,
        grid=(grid,),
        in_specs=[
            pl.BlockSpec((NC, blk, OUT_DIM), lambda i: (0, i, 0)),
            pl.BlockSpec((NC, blk, 1), lambda i: (0, i, 0)),
        ],
        out_specs=pl.BlockSpec((blk, OUT_DIM), lambda i: (i, 0)),
        out_shape=jax.ShapeDtypeStruct((N_NODES, OUT_DIM), jnp.float32),
    )(acc, den)


def kernel(x, edge_index, W_fc, W_attn):
    edge_index = edge_index.astype(jnp.int32)
    w_attn2 = jnp.concatenate(
        [W_attn[:OUT_DIM], W_attn[OUT_DIM:]], axis=1)
    z, s12 = _prep(x, W_fc, w_attn2)
    w_flat, src, dst, den = _scores(s12[:, 0], s12[:, 1], edge_index)
    acc = _rows(z, src, dst, w_flat)
    return _final(acc, den.reshape(NC, N_PAD, 1))
